# baseline (speedup 1.0000x reference)
import functools

import jax
import jax.numpy as jnp
from jax import lax
from jax.experimental import pallas as pl
from jax.experimental.pallas import tpu as pltpu
from jax.experimental.pallas import tpu_sc as plsc

_BM = 128
_BN = 512


def _expert_metadata(idx, num_experts, bm):
    counts = jnp.bincount(idx, length=num_experts).astype(jnp.int32)
    ends = jnp.cumsum(counts)
    starts = ends - counts
    fblk = (starts // bm).astype(jnp.int32)
    lblk = jnp.maximum(ends - 1, starts) // bm
    nblk = jnp.where(counts > 0, lblk - fblk + 1, 0).astype(jnp.int32)
    return fblk, nblk, starts.astype(jnp.int32), ends.astype(jnp.int32)


def _ffn_body(fb_ref, nb_ref, rs_ref, re_ref,
              w1_ref, w3_ref, w2_ref, xs_hbm, o_hbm,
              xs_v, o_v, w1c, w3c, w2c, sem, *, bm, num_n, num_steps):
    s = pl.program_id(0)
    par = lax.rem(s, 2)

    @pl.when(s == 0)
    def _load_x():
        pltpu.make_async_copy(xs_hbm, xs_v, sem).start()
        pltpu.make_async_copy(xs_hbm, xs_v, sem).wait()

    w1c[par] = w1_ref[0].astype(jnp.bfloat16)
    w3c[par] = w3_ref[0].astype(jnp.bfloat16)
    w2c[par] = w2_ref[0].astype(jnp.bfloat16)

    @pl.when(s == 1)
    def _zero():
        o_v[...] = jnp.zeros_like(o_v)

    @pl.when(s >= 1)
    def _compute():
        sd = s - 1
        e = sd // num_n
        pprev = lax.rem(sd, 2)
        w1 = w1c[pprev]
        w3 = w3c[pprev]
        w2b = w2c[pprev]
        fb = fb_ref[e]
        nb = nb_ref[e]
        lo = rs_ref[e]
        hi = re_ref[e]
        dn = (((1,), (1,)), ((), ()))

        def blk(b, carry):
            r0 = pl.multiple_of((fb + b) * bm, bm)
            xb = xs_v[pl.ds(r0, bm), :].astype(jnp.bfloat16)
            h1 = lax.dot_general(xb, w1, dn,
                                 preferred_element_type=jnp.float32)
            h3 = lax.dot_general(xb, w3, dn,
                                 preferred_element_type=jnp.float32)
            a = h1 * (1.0 / (1.0 + jnp.exp(-h1))) * h3
            y = lax.dot_general(a.astype(jnp.bfloat16), w2b, dn,
                                preferred_element_type=jnp.float32)
            rows = r0 + lax.broadcasted_iota(jnp.int32, (bm, 1), 0)
            msk = (rows >= lo) & (rows < hi)
            o_v[pl.ds(r0, bm), :] += jnp.where(msk, y, 0.0)
            return carry

        lax.fori_loop(0, nb, blk, 0)

    @pl.when(s == num_steps - 1)
    def _store():
        pltpu.make_async_copy(o_v, o_hbm, sem).start()
        pltpu.make_async_copy(o_v, o_hbm, sem).wait()


def _grouped_ffn(xs, w13, w2, fblk, nblk, rstart, rend, *, bm, bn):
    P, D = xs.shape
    E, two_i, _ = w13.shape
    inner = two_i // 2
    num_n = inner // bn
    num_steps = E * num_n + 1

    def w1_map(s, fb, nb, rs, re):
        t = jnp.minimum(s, E * num_n - 1)
        return (t // num_n, lax.rem(t, num_n), 0)

    def w3_map(s, fb, nb, rs, re):
        t = jnp.minimum(s, E * num_n - 1)
        return (t // num_n, lax.rem(t, num_n) + num_n, 0)

    def w2_map(s, fb, nb, rs, re):
        t = jnp.minimum(s, E * num_n - 1)
        return (t // num_n, 0, lax.rem(t, num_n))

    grid_spec = pltpu.PrefetchScalarGridSpec(
        num_scalar_prefetch=4,
        grid=(num_steps,),
        in_specs=[
            pl.BlockSpec((1, bn, D), w1_map),
            pl.BlockSpec((1, bn, D), w3_map),
            pl.BlockSpec((1, D, bn), w2_map),
            pl.BlockSpec(memory_space=pl.ANY),
        ],
        out_specs=pl.BlockSpec(memory_space=pl.ANY),
        scratch_shapes=[
            pltpu.VMEM((P, D), jnp.float32),
            pltpu.VMEM((P, D), jnp.float32),
            pltpu.VMEM((2, bn, D), jnp.bfloat16),
            pltpu.VMEM((2, bn, D), jnp.bfloat16),
            pltpu.VMEM((2, D, bn), jnp.bfloat16),
            pltpu.SemaphoreType.DMA,
        ],
    )
    return pl.pallas_call(
        functools.partial(_ffn_body, bm=bm, num_n=num_n,
                          num_steps=num_steps),
        grid_spec=grid_spec,
        out_shape=jax.ShapeDtypeStruct((P, D), jnp.float32),
        compiler_params=pltpu.CompilerParams(
            dimension_semantics=("arbitrary",),
        ),
    )(fblk, nblk, rstart, rend, w13, w13, w2, xs)


def _sc_gather(table, idxs):
    T, D = table.shape
    P = idxs.shape[0]
    info = plsc.get_sparse_core_info()
    nc, ns = info.num_cores, info.num_subcores
    nw = nc * ns
    bpw = P // nw
    ch = min(bpw, 64)
    nch = bpw // ch
    idx3 = idxs.reshape(nw, nch, ch)
    mesh = plsc.VectorSubcoreMesh(core_axis_name="c", subcore_axis_name="s")

    @functools.partial(
        pl.kernel,
        mesh=mesh,
        out_type=jax.ShapeDtypeStruct((P, D), table.dtype),
        scratch_types=[
            pltpu.VMEM((nch, ch), jnp.int32),
            pltpu.VMEM((ch, D), table.dtype),
            pltpu.SemaphoreType.DMA,
        ],
    )
    def gk(table_hbm, idx_hbm, out_hbm, idx_v, rows_v, sem):
        wid = lax.axis_index("s") * nc + lax.axis_index("c")
        base = wid * bpw
        pltpu.sync_copy(idx_hbm.at[wid], idx_v)
        for c in range(nch):
            pltpu.async_copy(table_hbm.at[idx_v.at[c]], rows_v, sem).wait()
            pltpu.sync_copy(rows_v, out_hbm.at[pl.ds(base + c * ch, ch)])

    return gk(table, idx3)


def kernel(x, expert_indices, w13, w2):
    M, D = x.shape
    E = w13.shape[0]
    A = expert_indices.shape[1]
    P = M * A

    idx = expert_indices.reshape(-1).astype(jnp.int32)
    order = jnp.argsort(idx, stable=True).astype(jnp.int32)
    rank = jnp.zeros((P,), jnp.int32).at[order].set(
        jnp.arange(P, dtype=jnp.int32))
    tok_sorted = order // A
    fblk, nblk, rstart, rend = _expert_metadata(idx, E, _BM)

    xs = _sc_gather(x, tok_sorted)
    ys = _grouped_ffn(xs, w13, w2, fblk, nblk, rstart, rend, bm=_BM, bn=_BN)
    return _sc_gather(ys, rank)
